# Initial kernel scaffold; baseline (speedup 1.0000x reference)
import jax
import jax.numpy as jnp
from jax.experimental import pallas as pl


def kernel(x, edge_index, edge_values, weight):
    raise NotImplementedError("write your pallas kernel here")


# speedup vs baseline: 5.8165x; 5.8165x over previous
import jax
import jax.numpy as jnp
from jax import lax
from jax.experimental import pallas as pl
from jax.experimental.pallas import tpu as pltpu
from jax.experimental.pallas import tpu_sc as plsc

_NC = 2
_NS = 16
_NW = _NC * _NS
_LANES = 16
_CH = 80
_SUP = 25


def _spmm_body(src_hbm, dst_hbm, val_hbm, x_hbm, zero_hbm, out_hbm,
               src_t, dst_t, val_t, rows, zsh):
    nsup = src_hbm.shape[1]
    ch = src_hbm.shape[3]
    sup_e = _SUP * ch
    d = x_hbm.shape[1]
    npad = out_hbm.shape[1]
    rpt = npad // _NS

    c = lax.axis_index("c")
    s = lax.axis_index("s")
    w = c * _NS + s

    pltpu.sync_copy(zero_hbm, zsh.at[pl.ds(s * rpt, rpt)])
    plsc.subcore_barrier()

    def super_chunk(t, carry):
        pltpu.sync_copy(src_hbm.at[w, t], src_t)
        pltpu.sync_copy(dst_hbm.at[w, t], dst_t)
        pltpu.sync_copy(val_hbm.at[pl.ds((w * nsup + t) * sup_e, sup_e)],
                        val_t.at[pl.ds(0, sup_e)])

        def chunk(i, carry1):
            pltpu.sync_copy(x_hbm.at[src_t.at[i]], rows)

            def edge(e, carry2):
                vv = val_t[pl.ds(i * ch + e, _LANES)]
                ve = lax.broadcast(vv[0], (_LANES,))
                for k in range(d // _LANES):
                    sl = pl.ds(k * _LANES, _LANES)
                    rows[e, sl] = rows[e, sl] * ve
                return carry2

            lax.fori_loop(0, ch, edge, 0, unroll=2)
            pltpu.sync_copy(rows, zsh.at[dst_t.at[i]], add=True)
            return carry1

        lax.fori_loop(0, _SUP, chunk, 0)
        return carry

    lax.fori_loop(0, nsup, super_chunk, 0)
    plsc.subcore_barrier()
    pltpu.sync_copy(zsh.at[pl.ds(s * rpt, rpt)],
                    out_hbm.at[c, pl.ds(s * rpt, rpt)])


def _spmm(src4, dst4, val, x, npad):
    n, d = x.shape
    zero = jnp.zeros((npad // _NS, d), jnp.float32)
    grid_kernel = pl.kernel(
        _spmm_body,
        out_type=jax.ShapeDtypeStruct((_NC, npad, d), jnp.float32),
        mesh=plsc.VectorSubcoreMesh(core_axis_name="c", subcore_axis_name="s"),
        scratch_types=[
            pltpu.VMEM((_SUP, _CH), jnp.int32),
            pltpu.VMEM((_SUP, _CH), jnp.int32),
            pltpu.VMEM((_SUP * _CH + _LANES,), jnp.float32),
            pltpu.VMEM((_CH, d), jnp.float32),
            pltpu.MemorySpace.VMEM_SHARED((npad, d), jnp.float32),
        ],
    )
    return grid_kernel(src4, dst4, val, x, zero)


def _matmul_body(z_ref, w_ref, o_ref):
    acc = z_ref[0] + z_ref[1]
    o_ref[...] = jnp.dot(acc, w_ref[...], preferred_element_type=jnp.float32)


def _sum_matmul(zp, weight, n):
    _, npad, d = zp.shape
    dout = weight.shape[1]
    bm = 1000
    return pl.pallas_call(
        _matmul_body,
        grid=(n // bm,),
        in_specs=[
            pl.BlockSpec((2, bm, d), lambda i: (0, i, 0)),
            pl.BlockSpec((d, dout), lambda i: (0, 0)),
        ],
        out_specs=pl.BlockSpec((bm, dout), lambda i: (i, 0)),
        out_shape=jax.ShapeDtypeStruct((n, dout), jnp.float32),
    )(zp, weight)


def kernel(x, edge_index, edge_values, weight):
    n = x.shape[0]
    e = edge_index.shape[1]
    npad = ((n + _NS * 8 - 1) // (_NS * 8)) * (_NS * 8)
    dst = edge_index[0]
    src = edge_index[1]
    nsup = e // (_NW * _SUP * _CH)
    src4 = src.reshape(_NW, nsup, _SUP, _CH)
    dst4 = dst.reshape(_NW, nsup, _SUP, _CH)
    zp = _spmm(src4, dst4, edge_values, x, npad)
    return _sum_matmul(zp, weight, n)


# speedup vs baseline: 8.4396x; 1.4510x over previous
import jax
import jax.numpy as jnp
from jax import lax
from jax.experimental import pallas as pl
from jax.experimental.pallas import tpu as pltpu
from jax.experimental.pallas import tpu_sc as plsc

_NC = 2
_NS = 16
_LANES = 16
_CH = 80
_NBUF = 3


def _spmm_body(src_hbm, dst_hbm, val_hbm, xr_hbm, zero_hbm, out_hbm,
               src_t, dst_t, val_t, rows, zsh, gsem, ssem):
    nchunk, ch = dst_t.shape
    ept = nchunk * ch
    dh = xr_hbm.shape[1]
    npad = out_hbm.shape[1]
    rpt = npad // _NS

    c = lax.axis_index("c")
    s = lax.axis_index("s")

    pltpu.sync_copy(src_hbm.at[pl.ds(s * ept, ept)],
                    src_t.at[pl.ds(0, ept)])
    pltpu.sync_copy(dst_hbm.at[s], dst_t)
    pltpu.sync_copy(val_hbm.at[pl.ds(s * ept, ept)],
                    val_t.at[pl.ds(0, ept)])
    pltpu.sync_copy(zero_hbm, zsh.at[pl.ds(s * rpt, rpt)])

    cvec = lax.broadcast(c, (_LANES,))

    def xform(m, carry):
        sl = pl.ds(m * _LANES, _LANES)
        v = src_t[sl]
        src_t[sl] = v + v + cvec
        return carry

    lax.fori_loop(0, ept // _LANES, xform, 0, unroll=4)
    plsc.subcore_barrier()

    def issue_gather(i, b):
        pltpu.async_copy(xr_hbm.at[src_t.at[pl.ds(i * ch, ch)]],
                         rows.at[b], gsem.at[b])

    def wait_gather(i, b):
        pltpu.make_async_copy(xr_hbm.at[src_t.at[pl.ds(i * ch, ch)]],
                              rows.at[b], gsem.at[b]).wait()

    def issue_scatter(i, b):
        pltpu.async_copy(rows.at[b], zsh.at[dst_t.at[i]], ssem.at[b],
                         add=True)

    def wait_scatter(i, b):
        pltpu.make_async_copy(rows.at[b], zsh.at[dst_t.at[i]],
                              ssem.at[b]).wait()

    def scale(i, b):
        rb = rows.at[b]

        def edge(e, carry):
            vv = val_t[pl.ds(i * ch + e, _LANES)]
            ve = lax.broadcast(vv[0], (_LANES,))
            for k in range(dh // _LANES):
                sl = pl.ds(k * _LANES, _LANES)
                rb[e, sl] = rb[e, sl] * ve
            return carry

        lax.fori_loop(0, ch, edge, 0, unroll=2)

    issue_gather(0, 0)
    issue_gather(1, 1)

    nmain = ((nchunk - 2) // _NBUF) * _NBUF

    def triple(j, carry):
        for p in range(_NBUF):
            i = j * _NBUF + p
            wait_gather(i, p)
            scale(i, p)
            if p == 0:
                @pl.when(j > 0)
                def _():
                    wait_scatter(i - 1, (p + 2) % _NBUF)
            else:
                wait_scatter(i - 1, (p + 2) % _NBUF)
            issue_gather(i + 2, (p + 2) % _NBUF)
            issue_scatter(i, p)
        return carry

    lax.fori_loop(0, nmain // _NBUF, triple, 0)

    for i in range(nmain, nchunk):
        b = i % _NBUF
        wait_gather(i, b)
        scale(i, b)
        wait_scatter(i - 1, (i + 2) % _NBUF)
        if i + 2 < nchunk:
            issue_gather(i + 2, (i + 2) % _NBUF)
        issue_scatter(i, b)
    wait_scatter(nchunk - 1, (nchunk - 1) % _NBUF)

    plsc.subcore_barrier()
    pltpu.sync_copy(zsh.at[pl.ds(s * rpt, rpt)],
                    out_hbm.at[c, pl.ds(s * rpt, rpt)])


def _spmm(src, dst3, val, xr, npad):
    n2, dh = xr.shape
    e = val.shape[0]
    nchunk = e // (_NS * _CH)
    ept = nchunk * _CH
    zero = jnp.zeros((npad // _NS, dh), jnp.float32)
    grid_kernel = pl.kernel(
        _spmm_body,
        out_type=jax.ShapeDtypeStruct((_NC, npad, dh), jnp.float32),
        mesh=plsc.VectorSubcoreMesh(core_axis_name="c", subcore_axis_name="s"),
        scratch_types=[
            pltpu.VMEM((ept + _LANES,), jnp.int32),
            pltpu.VMEM((nchunk, _CH), jnp.int32),
            pltpu.VMEM((ept + _LANES,), jnp.float32),
            pltpu.VMEM((_NBUF, _CH, dh), jnp.float32),
            pltpu.MemorySpace.VMEM_SHARED((npad, dh), jnp.float32),
            pltpu.SemaphoreType.DMA((_NBUF,)),
            pltpu.SemaphoreType.DMA((_NBUF,)),
        ],
        compiler_params=pltpu.CompilerParams(use_tc_tiling_on_sc=False),
    )
    return grid_kernel(src, dst3, val, xr, zero)


def _matmul_body(z_ref, w_ref, o_ref):
    dh = z_ref.shape[2]
    lo = jnp.dot(z_ref[0], w_ref[:dh, :], preferred_element_type=jnp.float32)
    hi = jnp.dot(z_ref[1], w_ref[dh:, :], preferred_element_type=jnp.float32)
    o_ref[...] = lo + hi


def _halves_matmul(zp, weight, n):
    _, npad, dh = zp.shape
    dout = weight.shape[1]
    bm = 1000
    return pl.pallas_call(
        _matmul_body,
        grid=(n // bm,),
        in_specs=[
            pl.BlockSpec((2, bm, dh), lambda i: (0, i, 0)),
            pl.BlockSpec((2 * dh, dout), lambda i: (0, 0)),
        ],
        out_specs=pl.BlockSpec((bm, dout), lambda i: (i, 0)),
        out_shape=jax.ShapeDtypeStruct((n, dout), jnp.float32),
    )(zp, weight)


def kernel(x, edge_index, edge_values, weight):
    n, d = x.shape
    e = edge_index.shape[1]
    dh = d // 2
    npad = ((n + _NS * 8 - 1) // (_NS * 8)) * (_NS * 8)
    dst = edge_index[0]
    src = edge_index[1]
    nchunk = e // (_NS * _CH)
    dst3 = dst.reshape(_NS, nchunk, _CH)
    xr = x.reshape(n * 2, dh)
    zp = _spmm(src, dst3, edge_values, xr, npad)
    return _halves_matmul(zp, weight, n)
